# initial kernel scaffold (unmeasured)
import jax
import jax.numpy as jnp
from jax import lax
from jax.experimental import pallas as pl
from jax.experimental.pallas import tpu as pltpu

N_DEV = 4


def kernel(x, w_mat):
    m, _ = x.shape
    _, n = w_mat.shape
    ch = m // N_DEV

    def body(x_ref, w_ref, out_ref, sbuf, rbuf, ag_ref,
             rs_send, rs_recv, ag_send, ag_recv):
        my = lax.axis_index("i")
        left = (my + N_DEV - 1) % N_DEV
        right = (my + 1) % N_DEV

        barrier = pltpu.get_barrier_semaphore()
        for nbr in (left, right):
            pl.semaphore_signal(barrier, inc=1, device_id=(nbr,),
                                device_id_type=pl.DeviceIdType.MESH)
        pl.semaphore_wait(barrier, 2)

        for c in range(N_DEV):
            out_ref[pl.ds(c * ch, ch), :] = jnp.dot(
                x_ref[pl.ds(c * ch, ch), :], w_ref[:, :],
                preferred_element_type=jnp.float32)

        for h in range(N_DEV - 1):
            c_send = (my + 2 * N_DEV - h - 1) % N_DEV
            acc = out_ref[pl.ds(c_send * ch, ch), :]
            if h > 0:
                acc = acc + rbuf[h - 1, :, :].astype(jnp.float32)
            sbuf[:, :] = acc.astype(jnp.bfloat16)
            rdma = pltpu.make_async_remote_copy(
                src_ref=sbuf,
                dst_ref=rbuf.at[h],
                send_sem=rs_send.at[h],
                recv_sem=rs_recv.at[h],
                device_id=(right,),
                device_id_type=pl.DeviceIdType.MESH,
            )
            rdma.start()
            rdma.wait()

        y_mine = (out_ref[pl.ds(my * ch, ch), :]
                  + rbuf[N_DEV - 2, :, :].astype(jnp.float32))
        ag_ref[pl.ds(my, 1), :, :] = y_mine.astype(jnp.bfloat16)[None]

        for h in range(N_DEV - 1):
            s_send = (my + N_DEV - h) % N_DEV
            rdma = pltpu.make_async_remote_copy(
                src_ref=ag_ref.at[s_send],
                dst_ref=ag_ref.at[s_send],
                send_sem=ag_send.at[h],
                recv_sem=ag_recv.at[h],
                device_id=(right,),
                device_id_type=pl.DeviceIdType.MESH,
            )
            rdma.start()
            rdma.wait()

        amax = jnp.max(jnp.abs(ag_ref[0, :, :].astype(jnp.float32)))
        for c in range(1, N_DEV):
            amax = jnp.maximum(
                amax, jnp.max(jnp.abs(ag_ref[c, :, :].astype(jnp.float32))))
        scale = amax / 127.0
        for c in range(N_DEV):
            y = ag_ref[c, :, :].astype(jnp.float32)
            q = jnp.clip(jnp.round(y / scale), -127.0, 127.0)
            out_ref[pl.ds(c * ch, ch), :] = q * scale

    return pl.pallas_call(
        body,
        out_shape=jax.ShapeDtypeStruct((m, n), jnp.float32),
        in_specs=[pl.BlockSpec(memory_space=pltpu.VMEM),
                  pl.BlockSpec(memory_space=pltpu.VMEM)],
        out_specs=pl.BlockSpec(memory_space=pltpu.VMEM),
        scratch_shapes=[
            pltpu.VMEM((ch, n), jnp.bfloat16),
            pltpu.VMEM((N_DEV - 1, ch, n), jnp.bfloat16),
            pltpu.VMEM((N_DEV, ch, n), jnp.bfloat16),
            pltpu.SemaphoreType.DMA((N_DEV - 1,)),
            pltpu.SemaphoreType.DMA((N_DEV - 1,)),
            pltpu.SemaphoreType.DMA((N_DEV - 1,)),
            pltpu.SemaphoreType.DMA((N_DEV - 1,)),
        ],
        compiler_params=pltpu.CompilerParams(collective_id=0),
    )(x, w_mat)


# baseline (device time: 346798 ns/iter reference)
import jax
import jax.numpy as jnp
from jax import lax
from jax.experimental import pallas as pl
from jax.experimental.pallas import tpu as pltpu

N_DEV = 4


def kernel(x, w_mat):
    m, _ = x.shape
    _, n = w_mat.shape
    ch = m // N_DEV

    def body(x_ref, w_ref, out_ref, sbuf, rbuf,
             rs_send, rs_recv, ag_send, ag_recv):
        my = lax.axis_index("i")
        left = (my + N_DEV - 1) % N_DEV
        right = (my + 1) % N_DEV

        barrier = pltpu.get_barrier_semaphore()
        for nbr in (left, right):
            pl.semaphore_signal(barrier, inc=1, device_id=(nbr,),
                                device_id_type=pl.DeviceIdType.MESH)
        pl.semaphore_wait(barrier, 2)

        blk = 256
        nb = ch // blk

        def stage_chunk(c, h, dst):
            for b in range(nb):
                part = jnp.dot(x_ref[pl.ds(c * ch + b * blk, blk), :],
                               w_ref[:, :],
                               preferred_element_type=jnp.float32)
                if h > 0:
                    part = part + rbuf[h - 1, pl.ds(b * blk, blk),
                                       :].astype(jnp.float32)
                dst[pl.ds(b * blk, blk), :] = part.astype(jnp.bfloat16)

        for h in range(N_DEV - 1):
            c_send = (my + 2 * N_DEV - h - 1) % N_DEV
            stage_chunk(c_send, h, sbuf)
            rdma = pltpu.make_async_remote_copy(
                src_ref=sbuf,
                dst_ref=rbuf.at[h],
                send_sem=rs_send.at[h],
                recv_sem=rs_recv.at[h],
                device_id=(right,),
                device_id_type=pl.DeviceIdType.MESH,
            )
            rdma.start()
            rdma.wait()

        for b in range(nb):
            part = jnp.dot(x_ref[pl.ds(my * ch + b * blk, blk), :],
                           w_ref[:, :], preferred_element_type=jnp.float32)
            part = part + rbuf[N_DEV - 2, pl.ds(b * blk, blk),
                               :].astype(jnp.float32)
            out_ref[pl.ds(my, 1), pl.ds(b * blk, blk), :] = (
                part.astype(jnp.bfloat16)[None])

        for h in range(N_DEV - 1):
            s_send = (my + N_DEV - h) % N_DEV
            rdma = pltpu.make_async_remote_copy(
                src_ref=out_ref.at[s_send],
                dst_ref=out_ref.at[s_send],
                send_sem=ag_send.at[h],
                recv_sem=ag_recv.at[h],
                device_id=(right,),
                device_id_type=pl.DeviceIdType.MESH,
            )
            rdma.start()
            rdma.wait()

        amax = jnp.float32(0.0)
        for c in range(N_DEV):
            for b in range(nb):
                blk_abs = jnp.abs(out_ref[c, pl.ds(b * blk, blk),
                                          :].astype(jnp.float32))
                amax = jnp.maximum(amax, jnp.max(blk_abs))
        scale = amax / 127.0
        for c in range(N_DEV):
            for b in range(nb):
                y = out_ref[c, pl.ds(b * blk, blk), :].astype(jnp.float32)
                q = jnp.clip(jnp.round(y / scale), -127.0, 127.0)
                out_ref[c, pl.ds(b * blk, blk), :] = (
                    q * scale).astype(jnp.bfloat16)

    out = pl.pallas_call(
        body,
        out_shape=jax.ShapeDtypeStruct((N_DEV, ch, n), jnp.bfloat16),
        in_specs=[pl.BlockSpec(memory_space=pltpu.VMEM),
                  pl.BlockSpec(memory_space=pltpu.VMEM)],
        out_specs=pl.BlockSpec(memory_space=pltpu.VMEM),
        scratch_shapes=[
            pltpu.VMEM((ch, n), jnp.bfloat16),
            pltpu.VMEM((N_DEV - 1, ch, n), jnp.bfloat16),
            pltpu.SemaphoreType.DMA((N_DEV - 1,)),
            pltpu.SemaphoreType.DMA((N_DEV - 1,)),
            pltpu.SemaphoreType.DMA((N_DEV - 1,)),
            pltpu.SemaphoreType.DMA((N_DEV - 1,)),
        ],
        compiler_params=pltpu.CompilerParams(
            collective_id=0, vmem_limit_bytes=36 * 1024 * 1024),
    )(x.astype(jnp.bfloat16), w_mat.astype(jnp.bfloat16))
    return out.reshape(m, n)


# device time: 212111 ns/iter; 1.6350x vs baseline; 1.6350x over previous
import jax
import jax.numpy as jnp
from jax import lax
from jax.experimental import pallas as pl
from jax.experimental.pallas import tpu as pltpu

N_DEV = 4


def kernel(x, w_mat):
    m, _ = x.shape
    _, n = w_mat.shape
    ch = m // N_DEV
    hh = ch // 2
    blk = 256

    def body(x_ref, w_ref, out_ref, sbuf_r, sbuf_l, rbuf_r, rbuf_l,
             rs_send_r, rs_recv_r, rs_send_l, rs_recv_l,
             ag_send_r, ag_recv_r, ag_send_l, ag_recv_l):
        my = lax.axis_index("i")
        left = (my + N_DEV - 1) % N_DEV
        right = (my + 1) % N_DEV

        barrier = pltpu.get_barrier_semaphore()
        for nbr in (left, right):
            pl.semaphore_signal(barrier, inc=1, device_id=(nbr,),
                                device_id_type=pl.DeviceIdType.MESH)
        pl.semaphore_wait(barrier, 2)

        def stage_half(c, row_off, h, rbuf, dst):
            for b in range(hh // blk):
                part = jnp.dot(
                    x_ref[pl.ds(c * ch + row_off + b * blk, blk), :],
                    w_ref[:, :], preferred_element_type=jnp.float32)
                if h > 0:
                    part = part + rbuf[h - 1, pl.ds(b * blk, blk),
                                       :].astype(jnp.float32)
                dst[pl.ds(b * blk, blk), :] = part.astype(jnp.bfloat16)

        for h in range(N_DEV - 1):
            cr = (my + 2 * N_DEV - h - 1) % N_DEV
            cl = (my + h + 1) % N_DEV
            stage_half(cr, 0, h, rbuf_r, sbuf_r)
            stage_half(cl, hh, h, rbuf_l, sbuf_l)
            rdma_r = pltpu.make_async_remote_copy(
                src_ref=sbuf_r, dst_ref=rbuf_r.at[h],
                send_sem=rs_send_r.at[h], recv_sem=rs_recv_r.at[h],
                device_id=(right,), device_id_type=pl.DeviceIdType.MESH)
            rdma_l = pltpu.make_async_remote_copy(
                src_ref=sbuf_l, dst_ref=rbuf_l.at[h],
                send_sem=rs_send_l.at[h], recv_sem=rs_recv_l.at[h],
                device_id=(left,), device_id_type=pl.DeviceIdType.MESH)
            rdma_r.start()
            rdma_l.start()
            rdma_r.wait()
            rdma_l.wait()

        for b in range(hh // blk):
            part = jnp.dot(x_ref[pl.ds(my * ch + b * blk, blk), :],
                           w_ref[:, :], preferred_element_type=jnp.float32)
            part = part + rbuf_r[N_DEV - 2, pl.ds(b * blk, blk),
                                 :].astype(jnp.float32)
            out_ref[pl.ds(my, 1), pl.ds(b * blk, blk), :] = (
                part.astype(jnp.bfloat16)[None])
        for b in range(hh // blk):
            part = jnp.dot(x_ref[pl.ds(my * ch + hh + b * blk, blk), :],
                           w_ref[:, :], preferred_element_type=jnp.float32)
            part = part + rbuf_l[N_DEV - 2, pl.ds(b * blk, blk),
                                 :].astype(jnp.float32)
            out_ref[pl.ds(my, 1), pl.ds(hh + b * blk, blk), :] = (
                part.astype(jnp.bfloat16)[None])

        for h in range(N_DEV - 1):
            sr = (my + N_DEV - h) % N_DEV
            sl = (my + h) % N_DEV
            rdma_r = pltpu.make_async_remote_copy(
                src_ref=out_ref.at[sr, pl.ds(0, hh)],
                dst_ref=out_ref.at[sr, pl.ds(0, hh)],
                send_sem=ag_send_r.at[h], recv_sem=ag_recv_r.at[h],
                device_id=(right,), device_id_type=pl.DeviceIdType.MESH)
            rdma_l = pltpu.make_async_remote_copy(
                src_ref=out_ref.at[sl, pl.ds(hh, hh)],
                dst_ref=out_ref.at[sl, pl.ds(hh, hh)],
                send_sem=ag_send_l.at[h], recv_sem=ag_recv_l.at[h],
                device_id=(left,), device_id_type=pl.DeviceIdType.MESH)
            rdma_r.start()
            rdma_l.start()
            rdma_r.wait()
            rdma_l.wait()

        amax = jnp.float32(0.0)
        for c in range(N_DEV):
            for b in range(ch // blk):
                blk_abs = jnp.abs(out_ref[c, pl.ds(b * blk, blk),
                                          :].astype(jnp.float32))
                amax = jnp.maximum(amax, jnp.max(blk_abs))
        scale = amax / 127.0
        for c in range(N_DEV):
            for b in range(ch // blk):
                y = out_ref[c, pl.ds(b * blk, blk), :].astype(jnp.float32)
                q = jnp.clip(jnp.round(y / scale), -127.0, 127.0)
                out_ref[c, pl.ds(b * blk, blk), :] = (
                    q * scale).astype(jnp.bfloat16)

    dma3 = pltpu.SemaphoreType.DMA((N_DEV - 1,))
    out = pl.pallas_call(
        body,
        out_shape=jax.ShapeDtypeStruct((N_DEV, ch, n), jnp.bfloat16),
        in_specs=[pl.BlockSpec(memory_space=pltpu.VMEM),
                  pl.BlockSpec(memory_space=pltpu.VMEM)],
        out_specs=pl.BlockSpec(memory_space=pltpu.VMEM),
        scratch_shapes=[
            pltpu.VMEM((hh, n), jnp.bfloat16),
            pltpu.VMEM((hh, n), jnp.bfloat16),
            pltpu.VMEM((N_DEV - 1, hh, n), jnp.bfloat16),
            pltpu.VMEM((N_DEV - 1, hh, n), jnp.bfloat16),
            dma3, dma3,
            dma3, dma3,
            dma3, dma3,
            dma3, dma3,
        ],
        compiler_params=pltpu.CompilerParams(
            collective_id=0, vmem_limit_bytes=36 * 1024 * 1024),
    )(x.astype(jnp.bfloat16), w_mat.astype(jnp.bfloat16))
    return out.reshape(m, n)


# device time: 195234 ns/iter; 1.7763x vs baseline; 1.0864x over previous
import jax
import jax.numpy as jnp
from jax import lax
from jax.experimental import pallas as pl
from jax.experimental.pallas import tpu as pltpu

N_DEV = 4


def kernel(x, w_mat):
    m, _ = x.shape
    _, n = w_mat.shape
    ch = m // N_DEV
    hh = ch // 2
    blk = 256

    def body(x_ref, w_ref, out_ref, sbuf_r, sbuf_l, pbuf_r, pbuf_l, am_ref,
             rs_send_r, rs_recv_r, rs_send_l, rs_recv_l,
             ag_send_r, ag_recv_r, ag_send_l, ag_recv_l,
             am_send_r, am_recv_r, am_send_l, am_recv_l):
        my = lax.axis_index("i")
        left = (my + N_DEV - 1) % N_DEV
        right = (my + 1) % N_DEV

        barrier = pltpu.get_barrier_semaphore()
        for nbr in (left, right):
            pl.semaphore_signal(barrier, inc=1, device_id=(nbr,),
                                device_id_type=pl.DeviceIdType.MESH)
        pl.semaphore_wait(barrier, 2)

        def dots_half(c, row_off, dst):
            for b in range(hh // blk):
                part = jnp.dot(
                    x_ref[pl.ds(c * ch + row_off + b * blk, blk), :],
                    w_ref[:, :], preferred_element_type=jnp.float32)
                dst[pl.ds(b * blk, blk), :] = part.astype(jnp.bfloat16)

        def add_half(pbuf, recv, dst):
            for b in range(hh // blk):
                s = pl.ds(b * blk, blk)
                acc = (pbuf[s, :].astype(jnp.float32)
                       + recv[s, :].astype(jnp.float32))
                dst[s, :] = acc.astype(jnp.bfloat16)

        dots_half((my + N_DEV - 1) % N_DEV, 0, sbuf_r)
        dots_half((my + 1) % N_DEV, hh, sbuf_l)
        for h in range(N_DEV - 1):
            cr = (my + 2 * N_DEV - h - 1) % N_DEV
            cl = (my + h + 1) % N_DEV
            rdma_r = pltpu.make_async_remote_copy(
                src_ref=sbuf_r, dst_ref=out_ref.at[cr, pl.ds(0, hh)],
                send_sem=rs_send_r.at[h], recv_sem=rs_recv_r.at[h],
                device_id=(right,), device_id_type=pl.DeviceIdType.MESH)
            rdma_l = pltpu.make_async_remote_copy(
                src_ref=sbuf_l, dst_ref=out_ref.at[cl, pl.ds(hh, hh)],
                send_sem=rs_send_l.at[h], recv_sem=rs_recv_l.at[h],
                device_id=(left,), device_id_type=pl.DeviceIdType.MESH)
            rdma_r.start()
            rdma_l.start()
            nr = (my + 2 * N_DEV - h - 2) % N_DEV if h < N_DEV - 2 else my
            nl = (my + h + 2) % N_DEV if h < N_DEV - 2 else my
            dots_half(nr, 0, pbuf_r)
            dots_half(nl, hh, pbuf_l)
            rdma_r.wait()
            rdma_l.wait()
            if h < N_DEV - 2:
                add_half(pbuf_r, out_ref.at[nr, pl.ds(0, hh)], sbuf_r)
                add_half(pbuf_l, out_ref.at[nl, pl.ds(hh, hh)], sbuf_l)
            else:
                add_half(pbuf_r, out_ref.at[my, pl.ds(0, hh)],
                         out_ref.at[my, pl.ds(0, hh)])
                add_half(pbuf_l, out_ref.at[my, pl.ds(hh, hh)],
                         out_ref.at[my, pl.ds(hh, hh)])

        amax_mine = jnp.float32(0.0)
        for b in range(ch // blk):
            t = jnp.abs(out_ref[my, pl.ds(b * blk, blk), :]
                        .astype(jnp.float32))
            amax_mine = jnp.maximum(amax_mine, jnp.max(t))
        am_ref[pl.ds(my, 1), :, :] = jnp.full((1, 8, 128), amax_mine,
                                              jnp.float32)
        am_r0 = pltpu.make_async_remote_copy(
            src_ref=am_ref.at[my], dst_ref=am_ref.at[my],
            send_sem=am_send_r.at[0], recv_sem=am_recv_r.at[0],
            device_id=(right,), device_id_type=pl.DeviceIdType.MESH)
        am_l0 = pltpu.make_async_remote_copy(
            src_ref=am_ref.at[my], dst_ref=am_ref.at[my],
            send_sem=am_send_l.at[0], recv_sem=am_recv_l.at[0],
            device_id=(left,), device_id_type=pl.DeviceIdType.MESH)
        am_r0.start()
        am_l0.start()
        am_r0.wait()
        am_l0.wait()
        sr1 = (my + N_DEV - 1) % N_DEV
        am_r1 = pltpu.make_async_remote_copy(
            src_ref=am_ref.at[sr1], dst_ref=am_ref.at[sr1],
            send_sem=am_send_r.at[1], recv_sem=am_recv_r.at[1],
            device_id=(right,), device_id_type=pl.DeviceIdType.MESH)
        am_r1.start()
        am_r1.wait()

        amax = jnp.max(am_ref[:, :, :])
        scale = amax / 127.0

        for b in range(ch // blk):
            y = out_ref[my, pl.ds(b * blk, blk), :].astype(jnp.float32)
            q = jnp.clip(jnp.round(y / scale), -127.0, 127.0)
            out_ref[my, pl.ds(b * blk, blk), :] = (
                q * scale).astype(jnp.bfloat16)

        for h in range(N_DEV - 1):
            sr = (my + N_DEV - h) % N_DEV
            sl = (my + h) % N_DEV
            rdma_r = pltpu.make_async_remote_copy(
                src_ref=out_ref.at[sr, pl.ds(0, hh)],
                dst_ref=out_ref.at[sr, pl.ds(0, hh)],
                send_sem=ag_send_r.at[h], recv_sem=ag_recv_r.at[h],
                device_id=(right,), device_id_type=pl.DeviceIdType.MESH)
            rdma_l = pltpu.make_async_remote_copy(
                src_ref=out_ref.at[sl, pl.ds(hh, hh)],
                dst_ref=out_ref.at[sl, pl.ds(hh, hh)],
                send_sem=ag_send_l.at[h], recv_sem=ag_recv_l.at[h],
                device_id=(left,), device_id_type=pl.DeviceIdType.MESH)
            rdma_r.start()
            rdma_l.start()
            rdma_r.wait()
            rdma_l.wait()

    dma1 = pltpu.SemaphoreType.DMA((1,))
    dma2 = pltpu.SemaphoreType.DMA((2,))
    dma3 = pltpu.SemaphoreType.DMA((N_DEV - 1,))
    out = pl.pallas_call(
        body,
        out_shape=jax.ShapeDtypeStruct((N_DEV, ch, n), jnp.bfloat16),
        in_specs=[pl.BlockSpec(memory_space=pltpu.VMEM),
                  pl.BlockSpec(memory_space=pltpu.VMEM)],
        out_specs=pl.BlockSpec(memory_space=pltpu.VMEM),
        scratch_shapes=[
            pltpu.VMEM((hh, n), jnp.bfloat16),
            pltpu.VMEM((hh, n), jnp.bfloat16),
            pltpu.VMEM((hh, n), jnp.bfloat16),
            pltpu.VMEM((hh, n), jnp.bfloat16),
            pltpu.VMEM((N_DEV, 8, 128), jnp.float32),
            dma3, dma3,
            dma3, dma3,
            dma3, dma3,
            dma3, dma3,
            dma2, dma2,
            dma1, dma1,
        ],
        compiler_params=pltpu.CompilerParams(
            collective_id=0, vmem_limit_bytes=36 * 1024 * 1024),
    )(x.astype(jnp.bfloat16), w_mat.astype(jnp.bfloat16))
    return out.reshape(m, n)


# device time: 182615 ns/iter; 1.8991x vs baseline; 1.0691x over previous
import jax
import jax.numpy as jnp
from jax import lax
from jax.experimental import pallas as pl
from jax.experimental.pallas import tpu as pltpu

N_DEV = 4


def kernel(x, w_mat):
    m, _ = x.shape
    _, n = w_mat.shape
    ch = m // N_DEV
    hh = ch // 2
    sg = 256

    def body(x_ref, w_ref, out_ref, sbuf_r, sbuf_l, pbuf_r, pbuf_l, am_ref,
             rs_send_r, rs_recv_r, rs_send_l, rs_recv_l,
             ag_send_r, ag_recv_r, ag_send_l, ag_recv_l,
             am_send, am_recv):
        my = lax.axis_index("i")
        left = (my + N_DEV - 1) % N_DEV
        right = (my + 1) % N_DEV

        def rcopy(src, dst, ss, rs, dev):
            return pltpu.make_async_remote_copy(
                src_ref=src, dst_ref=dst, send_sem=ss, recv_sem=rs,
                device_id=(dev,), device_id_type=pl.DeviceIdType.MESH)

        barrier = pltpu.get_barrier_semaphore()
        for nbr in (left, right):
            pl.semaphore_signal(barrier, inc=1, device_id=(nbr,),
                                device_id_type=pl.DeviceIdType.MESH)
        pl.semaphore_wait(barrier, 2)

        def dot_seg(c, row_off, s, dst):
            part = jnp.dot(
                x_ref[pl.ds(c * ch + row_off + s * sg, sg), :],
                w_ref[:, :], preferred_element_type=jnp.float32)
            dst[pl.ds(s * sg, sg), :] = part.astype(jnp.bfloat16)

        def rseg(s):
            return pl.ds(s * sg, sg)

        def lseg(s):
            return pl.ds(hh + s * sg, sg)

        for s in range(2):
            dot_seg((my + N_DEV - 1) % N_DEV, 0, s, sbuf_r)
            dot_seg((my + 1) % N_DEV, hh, s, sbuf_l)
        send_r = [None, None]
        send_l = [None, None]
        cr0 = (my + N_DEV - 1) % N_DEV
        cl0 = (my + 1) % N_DEV
        for s in range(2):
            send_r[s] = rcopy(sbuf_r.at[rseg(s)], out_ref.at[cr0, rseg(s)],
                              rs_send_r.at[0, s], rs_recv_r.at[0, s], right)
            send_l[s] = rcopy(sbuf_l.at[rseg(s)],
                              out_ref.at[cl0, lseg(s)],
                              rs_send_l.at[0, s], rs_recv_l.at[0, s], left)
            send_r[s].start()
            send_l[s].start()

        amax_mine = jnp.float32(0.0)
        for h in range(N_DEV - 1):
            nr = (my + 2 * N_DEV - h - 2) % N_DEV if h < N_DEV - 2 else my
            nl = (my + h + 2) % N_DEV if h < N_DEV - 2 else my
            for s in range(2):
                dot_seg(nr, 0, s, pbuf_r)
                dot_seg(nl, hh, s, pbuf_l)
            for s in range(2):
                rcopy(sbuf_r.at[rseg(s)], out_ref.at[nr, rseg(s)],
                      rs_send_r.at[h, s], rs_recv_r.at[h, s],
                      right).wait_recv()
                rcopy(sbuf_l.at[rseg(s)], out_ref.at[nl, lseg(s)],
                      rs_send_l.at[h, s], rs_recv_l.at[h, s],
                      left).wait_recv()
                if h < N_DEV - 2:
                    send_r[s].wait_send()
                    send_l[s].wait_send()
                    acc_r = (pbuf_r[rseg(s), :].astype(jnp.float32)
                             + out_ref[nr, rseg(s), :].astype(jnp.float32))
                    sbuf_r[rseg(s), :] = acc_r.astype(jnp.bfloat16)
                    acc_l = (pbuf_l[rseg(s), :].astype(jnp.float32)
                             + out_ref[nl, lseg(s), :].astype(jnp.float32))
                    sbuf_l[rseg(s), :] = acc_l.astype(jnp.bfloat16)
                    send_r[s] = rcopy(
                        sbuf_r.at[rseg(s)], out_ref.at[nr, rseg(s)],
                        rs_send_r.at[h + 1, s], rs_recv_r.at[h + 1, s],
                        right)
                    send_l[s] = rcopy(
                        sbuf_l.at[rseg(s)], out_ref.at[nl, lseg(s)],
                        rs_send_l.at[h + 1, s], rs_recv_l.at[h + 1, s],
                        left)
                    send_r[s].start()
                    send_l[s].start()
                else:
                    acc_r = (pbuf_r[rseg(s), :].astype(jnp.float32)
                             + out_ref[my, rseg(s), :].astype(jnp.float32))
                    y_r = acc_r.astype(jnp.bfloat16)
                    out_ref[my, rseg(s), :] = y_r
                    amax_mine = jnp.maximum(
                        amax_mine,
                        jnp.max(jnp.abs(y_r.astype(jnp.float32))))
                    acc_l = (pbuf_l[rseg(s), :].astype(jnp.float32)
                             + out_ref[my, lseg(s), :].astype(jnp.float32))
                    y_l = acc_l.astype(jnp.bfloat16)
                    out_ref[my, lseg(s), :] = y_l
                    amax_mine = jnp.maximum(
                        amax_mine,
                        jnp.max(jnp.abs(y_l.astype(jnp.float32))))
        send_r[0].wait_send()
        send_l[0].wait_send()
        send_r[1].wait_send()
        send_l[1].wait_send()

        am_ref[pl.ds(my, 1), :, :] = jnp.full((1, 8, 128), amax_mine,
                                              jnp.float32)
        am_ops = []
        for k in range(1, N_DEV):
            op = rcopy(am_ref.at[my], am_ref.at[my],
                       am_send.at[k - 1], am_recv.at[k - 1],
                       (my + k) % N_DEV)
            op.start()
            am_ops.append(op)
        for op in am_ops:
            op.wait()

        amax = jnp.max(am_ref[:, :, :])
        scale = amax / 127.0

        for b in range(ch // sg):
            y = out_ref[my, pl.ds(b * sg, sg), :].astype(jnp.float32)
            q = jnp.clip(jnp.round(y / scale), -127.0, 127.0)
            out_ref[my, pl.ds(b * sg, sg), :] = (
                q * scale).astype(jnp.bfloat16)

        ag_ops = []

        def ag_start(h, s):
            sr = (my + N_DEV - h) % N_DEV
            sl = (my + h) % N_DEV
            opr = rcopy(out_ref.at[sr, rseg(s)], out_ref.at[sr, rseg(s)],
                        ag_send_r.at[h, s], ag_recv_r.at[h, s], right)
            opl = rcopy(out_ref.at[sl, lseg(s)], out_ref.at[sl, lseg(s)],
                        ag_send_l.at[h, s], ag_recv_l.at[h, s], left)
            opr.start()
            opl.start()
            ag_ops.extend([opr, opl])

        def ag_wait_recv(h, s):
            rr = (my + 2 * N_DEV - h - 1) % N_DEV
            rl = (my + h + 1) % N_DEV
            rcopy(out_ref.at[rr, rseg(s)], out_ref.at[rr, rseg(s)],
                  ag_send_r.at[h, s], ag_recv_r.at[h, s], right).wait_recv()
            rcopy(out_ref.at[rl, lseg(s)], out_ref.at[rl, lseg(s)],
                  ag_send_l.at[h, s], ag_recv_l.at[h, s], left).wait_recv()

        ag_start(0, 0)
        ag_start(0, 1)
        for h in range(1, N_DEV - 1):
            for s in range(2):
                ag_wait_recv(h - 1, s)
                ag_start(h, s)
        for s in range(2):
            ag_wait_recv(N_DEV - 2, s)
        for op in ag_ops:
            op.wait_send()

    dma3 = pltpu.SemaphoreType.DMA((N_DEV - 1,))
    dma32 = pltpu.SemaphoreType.DMA((N_DEV - 1, 2))
    out = pl.pallas_call(
        body,
        out_shape=jax.ShapeDtypeStruct((N_DEV, ch, n), jnp.bfloat16),
        in_specs=[pl.BlockSpec(memory_space=pltpu.VMEM),
                  pl.BlockSpec(memory_space=pltpu.VMEM)],
        out_specs=pl.BlockSpec(memory_space=pltpu.VMEM),
        scratch_shapes=[
            pltpu.VMEM((hh, n), jnp.bfloat16),
            pltpu.VMEM((hh, n), jnp.bfloat16),
            pltpu.VMEM((hh, n), jnp.bfloat16),
            pltpu.VMEM((hh, n), jnp.bfloat16),
            pltpu.VMEM((N_DEV, 8, 128), jnp.float32),
            dma32, dma32,
            dma32, dma32,
            dma32, dma32,
            dma32, dma32,
            dma3, dma3,
        ],
        compiler_params=pltpu.CompilerParams(
            collective_id=0, vmem_limit_bytes=36 * 1024 * 1024),
    )(x.astype(jnp.bfloat16), w_mat.astype(jnp.bfloat16))
    return out.reshape(m, n)
